# Initial kernel scaffold; baseline (speedup 1.0000x reference)
import jax
import jax.numpy as jnp
from jax.experimental import pallas as pl


def kernel(sudoku, answers, params):
    raise NotImplementedError("write your pallas kernel here")


# speedup vs baseline: 1.7379x; 1.7379x over previous
import functools

import jax
import jax.numpy as jnp
from jax import lax
from jax.experimental import pallas as pl
from jax.experimental.pallas import tpu as pltpu

_EMB = 16
_H = 16
_MLP = 96
_STEPS = 25
_BSZ = 256
_BB = 16


def _mm(a, b):
    return jax.lax.dot_general(a, b, (((a.ndim - 1,), (0,)), ((), ())),
                               preferred_element_type=jnp.float32)


def _grid_transpose(v, bb, f):
    return v.reshape(bb, 8, 8, f).transpose(0, 2, 1, 3).reshape(bb * 64, f)


def _rrn_block(sud_ref, ans_ref,
               emb_in_ref, emb_r_ref, emb_c_ref,
               w0s_ref, w0r_ref, w0c_ref, b0_ref, w1_ref, b1_ref,
               w2_ref, b2_ref, w3_ref, b3_ref,
               f0a_ref, f0b_ref, fb0_ref, f1_ref, fbb1_ref,
               f2_ref, fbb2_ref, f3_ref, fbb3_ref,
               g0a_ref, g0b_ref, gb0_ref, g1_ref, gb1_ref,
               g2_ref, gb2_ref, g3_ref, gb3_ref,
               wih_ref, whh_ref, blstm_ref,
               outw_ref, outb_ref,
               out_ref, loss_ref):
    n = sud_ref.shape[0]
    bb = n // 64
    f32 = jnp.float32

    sud = sud_ref[...]
    oh16 = (jnp.broadcast_to(sud, (n, 16))
            == lax.broadcasted_iota(jnp.int32, (n, 16), 1)).astype(f32)
    e_s = _mm(oh16, emb_in_ref[...])

    idx = lax.broadcasted_iota(jnp.int32, (64, 8), 0)
    lane8 = lax.broadcasted_iota(jnp.int32, (64, 8), 1)
    ohr = ((idx % 8) == lane8).astype(f32)
    ohc = ((idx // 8) == lane8).astype(f32)
    er = _mm(ohr, emb_r_ref[...])
    ec = _mm(ohc, emb_c_ref[...])
    nodepre = _mm(er, w0r_ref[...]) + _mm(ec, w0c_ref[...]) + b0_ref[...]
    nodepre_n = jnp.broadcast_to(nodepre.reshape(1, 64, _MLP),
                                 (bb, 64, _MLP)).reshape(n, _MLP)

    xh = jax.nn.relu(_mm(e_s, w0s_ref[...]) + nodepre_n)
    xh = jax.nn.relu(_mm(xh, w1_ref[...]) + b1_ref[...])
    xh = jax.nn.relu(_mm(xh, w2_ref[...]) + b2_ref[...])
    x = _mm(xh, w3_ref[...]) + b3_ref[...]

    xg = _mm(x, g0a_ref[...]) + gb0_ref[...]
    lab = ans_ref[...] - 1
    oh_lab = (jnp.broadcast_to(lab, (n, 8))
              == lax.broadcasted_iota(jnp.int32, (n, 8), 1)).astype(f32)

    i0 = lax.broadcasted_iota(jnp.int32, (n, 8, 1), 0)
    j0 = lax.broadcasted_iota(jnp.int32, (n, 8, 1), 1)
    pmask = ((i0 % 8) != j0).astype(f32)

    f0a, f0b = f0a_ref[...], f0b_ref[...]
    fb0 = fb0_ref[...]
    f1, fbb1 = f1_ref[...], fbb1_ref[...]
    f2, fbb2 = f2_ref[...], fbb2_ref[...]
    f3, fbb3 = f3_ref[...], fbb3_ref[...]
    g0b = g0b_ref[...]
    g1, gb1 = g1_ref[...], gb1_ref[...]
    g2, gb2 = g2_ref[...], gb2_ref[...]
    g3, gb3 = g3_ref[...], gb3_ref[...]
    wih, whh, blstm = wih_ref[...], whh_ref[...], blstm_ref[...]
    outw, outb = outw_ref[...], outb_ref[...]

    def pair_messages(a, bmat):
        a_rep = jnp.broadcast_to(a.reshape(n, 1, _MLP),
                                 (n, 8, _MLP)).reshape(n * 8, _MLP)
        b_tile = jnp.broadcast_to(bmat.reshape(bb * 8, 1, 8, _MLP),
                                  (bb * 8, 8, 8, _MLP)).reshape(n * 8, _MLP)
        p = jax.nn.relu(a_rep + b_tile)
        p = jax.nn.relu(_mm(p, f1) + fbb1)
        p = jax.nn.relu(_mm(p, f2) + fbb2)
        msg = _mm(p, f3) + fbb3
        msg = msg.reshape(n, 8, _H) * pmask
        return jnp.sum(msg, axis=1)

    def step(_, carry):
        h, s, acc = carry
        a = _mm(h, f0a) + fb0
        bmat = _mm(h, f0b)
        m_row = pair_messages(a, bmat)
        at = _grid_transpose(a, bb, _MLP)
        bt = _grid_transpose(bmat, bb, _MLP)
        m_colt = pair_messages(at, bt)
        m = m_row + _grid_transpose(m_colt, bb, _H)

        q = jax.nn.relu(xg + _mm(m, g0b))
        q = jax.nn.relu(_mm(q, g1) + gb1)
        q = jax.nn.relu(_mm(q, g2) + gb2)
        lstm_inp = _mm(q, g3) + gb3

        gates = _mm(lstm_inp, wih) + _mm(h, whh) + blstm
        i_g = gates[:, 0:16]
        f_g = gates[:, 16:32]
        g_g = gates[:, 32:48]
        o_g = gates[:, 48:64]
        c = jax.nn.sigmoid(f_g) * s + jax.nn.sigmoid(i_g) * jnp.tanh(g_g)
        h_new = jax.nn.sigmoid(o_g) * jnp.tanh(c)

        logits = _mm(h_new, outw) + outb
        mx = jnp.max(logits, axis=1, keepdims=True)
        z = logits - mx
        lse = jnp.log(jnp.sum(jnp.exp(z), axis=1, keepdims=True))
        z_lab = jnp.sum(z * oh_lab, axis=1, keepdims=True)
        acc = acc + jnp.sum(lse - z_lab)
        return h_new, c, acc

    h0 = x
    s0 = jnp.zeros((n, _H), dtype=f32)
    h, s, acc = lax.fori_loop(0, _STEPS, step, (h0, s0, jnp.float32(0.0)))

    out_ref[...] = (_mm(h, outw) + outb).reshape(bb, 64, 8)
    loss_ref[...] = (acc / jnp.float32(_BSZ * 64 * _STEPS)).reshape(1, 1, 1)


def _full(shape):
    nd = len(shape)
    return pl.BlockSpec(shape, lambda i, _nd=nd: (0,) * _nd)


@jax.jit
def kernel(sudoku, answers, params):
    f32 = jnp.float32
    p = params

    def t(w):
        return jnp.asarray(w, f32).T

    emb_in = jnp.zeros((16, _EMB), f32).at[:9, :].set(p['embed_input'])
    (w0, b0), (w1, b1), (w2, b2), (w3, b3) = p['mlp']
    (fw0, fb0), (fw1, fb1), (fw2, fb2), (fw3, fb3) = p['f']
    (g0, gb0), (g1, gb1), (g2, gb2), (g3, gb3) = p['mlp2']
    w0t = t(w0)
    f0t = t(fw0)
    g0t = t(g0)

    row = lambda b: jnp.asarray(b, f32).reshape(1, -1)
    operands = [
        sudoku.astype(jnp.int32).reshape(-1, 1),
        answers.astype(jnp.int32).reshape(-1, 1),
        emb_in, p['embed_rows'], p['embed_cols'],
        w0t[0:16], w0t[16:32], w0t[32:48], row(b0), t(w1), row(b1),
        t(w2), row(b2), t(w3), row(b3),
        f0t[0:16], f0t[16:32], row(fb0), t(fw1), row(fb1),
        t(fw2), row(fb2), t(fw3), row(fb3),
        g0t[0:16], g0t[16:32], row(gb0), t(g1), row(gb1),
        t(g2), row(gb2), t(g3), row(gb3),
        t(p['lstm_W_ih']), t(p['lstm_W_hh']),
        row(p['lstm_b_ih'] + p['lstm_b_hh']),
        t(p['out_W']), row(p['out_b']),
    ]

    nblocks = _BSZ // _BB
    in_specs = [
        pl.BlockSpec((_BB * 64, 1), lambda i: (i, 0)),
        pl.BlockSpec((_BB * 64, 1), lambda i: (i, 0)),
    ] + [_full(op.shape) for op in operands[2:]]

    out, loss_parts = pl.pallas_call(
        _rrn_block,
        grid=(nblocks,),
        in_specs=in_specs,
        out_specs=[
            pl.BlockSpec((_BB, 64, 8), lambda i: (i, 0, 0)),
            pl.BlockSpec((1, 1, 1), lambda i: (i, 0, 0)),
        ],
        out_shape=[
            jax.ShapeDtypeStruct((_BSZ, 64, 8), f32),
            jax.ShapeDtypeStruct((nblocks, 1, 1), f32),
        ],
        compiler_params=pltpu.CompilerParams(
            dimension_semantics=("parallel",),
        ),
    )(*operands)
    return out, jnp.sum(loss_parts)
